# baseline (speedup 1.0000x reference)
import functools

import jax
import jax.numpy as jnp
from jax import lax
from jax.experimental import pallas as pl
from jax.experimental.pallas import tpu as pltpu
from jax.experimental.pallas import tpu_sc as plsc

_R, _C = 8192, 512
_S_ROWS = 2048
_T_ROWS = _R - _S_ROWS
_TC_BLOCK_R = 512
_NC, _NS, _L = 2, 16, 16
_NW = _NC * _NS
_ROWS_W = _S_ROWS // _NW
_CHUNK_R = 16
_NCHUNKS = _ROWS_W // _CHUNK_R
_UNROLL = 4
_LN2 = 0.6931471805599453

_D0 = 89.76645166961427
_D1 = -2.806980531444208
_D2 = 1.455194772067026
_D3 = -0.440502738630688
_D4 = 0.05545931374210546

_mesh = plsc.VectorSubcoreMesh(core_axis_name="c", subcore_axis_name="s")


def _neg_log_clipped(q):
    bits = lax.bitcast_convert_type(q, jnp.int32)
    e = (bits >> 23).astype(jnp.float32)
    m = lax.bitcast_convert_type((bits & 0x007FFFFF) | 0x3F800000, jnp.float32)
    nl_m = _D0 + m * (_D1 + m * (_D2 + m * (_D3 + m * _D4)))
    nl = e * (-_LN2) + nl_m
    return jnp.where(q > 0.0, nl, 100.0)


@functools.partial(
    pl.kernel,
    mesh=_mesh,
    out_type=jax.ShapeDtypeStruct((_NW, 4, _L), jnp.float32),
    scratch_types=[
        pltpu.VMEM((2, _CHUNK_R, _C), jnp.float32),
        pltpu.VMEM((2, _CHUNK_R, _C), jnp.float32),
        pltpu.VMEM((2, _CHUNK_R, _C), jnp.float32),
        pltpu.VMEM((4, _L), jnp.float32),
        pltpu.SemaphoreType.DMA,
        pltpu.SemaphoreType.DMA,
    ],
    compiler_params=pltpu.CompilerParams(use_tc_tiling_on_sc=True),
)
def _pass1(pred_h, gt_h, mask_h, out_h, pb, gb, mb, acc, s0, s1):
    wid = lax.axis_index("c") * _NS + lax.axis_index("s")
    base = wid * _ROWS_W
    sems = (s0, s1)

    def start(c):
        b = c % 2
        row = base + c * _CHUNK_R
        return [
            pltpu.async_copy(h.at[pl.ds(row, _CHUNK_R), :], buf.at[b], sems[b])
            for h, buf in ((pred_h, pb), (gt_h, gb), (mask_h, mb))
        ]

    zeros = jnp.zeros((_L,), jnp.float32)
    carry = (zeros, zeros, zeros, zeros)
    pend = start(0)
    for c in range(_NCHUNKS):
        nxt = start(c + 1) if c + 1 < _NCHUNKS else None
        for h in pend:
            h.wait()
        b = c % 2

        def row_body(r, cr0, b=b):
            def slice_body(i, cr, b=b, r=r):
                ps, ts, pc, mc = cr
                for u in range(_UNROLL):
                    sl = pl.ds(i * (_L * _UNROLL) + u * _L, _L)
                    p = pb[b, r, sl]
                    g = gb[b, r, sl]
                    m = mb[b, r, sl]
                    q = jnp.where(g > 0.5, p, 1.0 - p)
                    l = _neg_log_clipped(q)
                    gm = g * m
                    ps = ps + l * gm
                    ts = ts + l * m
                    pc = pc + gm
                    mc = mc + m
                return (ps, ts, pc, mc)

            return lax.fori_loop(0, _C // (_L * _UNROLL), slice_body, cr0)

        carry = lax.fori_loop(0, _CHUNK_R, row_body, carry)
        pend = nxt
    ps, ts, pc, mc = carry
    acc[0, :] = ps
    acc[1, :] = ts
    acc[2, :] = pc
    acc[3, :] = mc
    pltpu.sync_copy(acc, out_h.at[wid])


def _fold8(x):
    acc = x[0:8]
    for i in range(1, _TC_BLOCK_R // 8):
        acc = acc + x[i * 8:(i + 1) * 8]
    return acc


def _tc_body(p_ref, g_ref, m_ref, o_ref):
    p = p_ref[...]
    g = g_ref[...]
    m = m_ref[...]
    q = jnp.where(g > 0.5, p, 1.0 - p)
    bits = lax.bitcast_convert_type(q, jnp.int32)
    e = (bits >> 23).astype(jnp.float32)
    mant = lax.bitcast_convert_type((bits & 0x007FFFFF) | 0x3F800000, jnp.float32)
    nl_m = _D0 + mant * (_D1 + mant * (_D2 + mant * (_D3 + mant * _D4)))
    l = jnp.where(q > 0.0, e * (-_LN2) + nl_m, 100.0)
    gm = g * m
    w8 = jnp.full((8, _TC_BLOCK_R), 0.125, jnp.float32)
    o_ref[...] = jnp.concatenate(
        [w8 @ (l * gm), w8 @ (l * m), w8 @ gm, w8 @ m], axis=0
    )


def _thr_body(p_ref, g_ref, m_ref, t_ref, o_ref):
    t = t_ref[0, 0]
    p = p_ref[...]
    g = g_ref[...]
    m = m_ref[...]
    q = jnp.where(g > 0.5, p, 1.0 - p)
    bits = lax.bitcast_convert_type(q, jnp.int32)
    e = (bits >> 23).astype(jnp.float32)
    mant = lax.bitcast_convert_type((bits & 0x007FFFFF) | 0x3F800000, jnp.float32)
    nl_m = _D0 + mant * (_D1 + mant * (_D2 + mant * (_D3 + mant * _D4)))
    l = jnp.where(q > 0.0, e * (-_LN2) + nl_m, 100.0)
    v = l * ((1.0 - g) * m)
    above = (v > t).astype(jnp.float32)
    o_ref[...] = jnp.concatenate([_fold8(above), _fold8(v * above)], axis=0)


_thr_grid = _R // _TC_BLOCK_R
_thr_pass = pl.pallas_call(
    _thr_body,
    grid=(_thr_grid,),
    in_specs=[pl.BlockSpec((_TC_BLOCK_R, _C), lambda i: (i, 0))] * 3
    + [pl.BlockSpec(memory_space=pltpu.SMEM)],
    out_specs=pl.BlockSpec((16, _C), lambda i: (i, 0)),
    out_shape=jax.ShapeDtypeStruct((_thr_grid * 16, _C), jnp.float32),
)


def _count_sum_above(pf, gf, mf, thresh):
    parts = _thr_pass(pf, gf, mf, thresh.reshape(1, 1))
    r = parts.reshape(_thr_grid, 2, 8, _C).sum(axis=(0, 2, 3))
    return r[0], r[1]


_tc_grid = _T_ROWS // _TC_BLOCK_R
_tc_off = _S_ROWS // _TC_BLOCK_R
_tc_pass = pl.pallas_call(
    _tc_body,
    grid=(_tc_grid,),
    in_specs=[pl.BlockSpec((_TC_BLOCK_R, _C), lambda i: (i + _tc_off, 0))] * 3,
    out_specs=pl.BlockSpec((32, _C), lambda i: (i, 0)),
    out_shape=jax.ShapeDtypeStruct((_tc_grid * 32, _C), jnp.float32),
)


def kernel(pred, gt, mask):
    pf = pred.reshape(_R, _C)
    gf = gt.reshape(_R, _C)
    mf = mask.reshape(_R, _C)
    parts = _pass1(pf, gf, mf)
    tc_parts = _tc_pass(pf, gf, mf)
    t = parts.sum(axis=(0, 2)) + tc_parts.reshape(_tc_grid, 4, 8, _C).sum(
        axis=(0, 2, 3)
    )
    pos_loss, tot_loss, no_pos, mask_cnt = t[0], t[1], t[2], t[3]
    neg_loss = tot_loss - pos_loss
    neg_cnt = mask_cnt - no_pos
    no_neg = jnp.minimum(no_pos * 3.0, neg_cnt)

    def _topk_all(_):
        return neg_loss

    def _topk_search(_):
        k = no_neg

        def body(st):
            lo, hi = st
            mid = (lo + hi) // 2
            c, _ = _count_sum_above(pf, gf, mf, lax.bitcast_convert_type(mid, jnp.float32))
            ok = c <= k
            return (jnp.where(ok, lo, mid + 1), jnp.where(ok, mid, hi))

        lo, _ = lax.while_loop(
            lambda st: st[0] < st[1],
            body,
            (jnp.int32(0), jnp.int32(0x42C80000)),
        )
        tstar = lax.bitcast_convert_type(lo, jnp.float32)
        c, s = _count_sum_above(pf, gf, mf, tstar)
        return s + (k - c) * tstar

    topk = lax.cond(no_neg >= neg_cnt, _topk_all, _topk_search, None)
    return (pos_loss + topk) / (no_neg + no_pos + 1e-6)


# speedup vs baseline: 1.0166x; 1.0166x over previous
import functools

import jax
import jax.numpy as jnp
from jax import lax
from jax.experimental import pallas as pl
from jax.experimental.pallas import tpu as pltpu
from jax.experimental.pallas import tpu_sc as plsc

_R, _C = 8192, 512
_S_ROWS = 2560
_T_ROWS = _R - _S_ROWS
_TC_BLOCK_R = 512
_NC, _NS, _L = 2, 16, 16
_NW = _NC * _NS
_ROWS_W = _S_ROWS // _NW
_CHUNK_R = 16
_NCHUNKS = _ROWS_W // _CHUNK_R
_UNROLL = 4
_LN2 = 0.6931471805599453

_D0 = 89.76645166961427
_D1 = -2.806980531444208
_D2 = 1.455194772067026
_D3 = -0.440502738630688
_D4 = 0.05545931374210546

_mesh = plsc.VectorSubcoreMesh(core_axis_name="c", subcore_axis_name="s")


def _neg_log_clipped(q):
    bits = lax.bitcast_convert_type(q, jnp.int32)
    e = (bits >> 23).astype(jnp.float32)
    m = lax.bitcast_convert_type((bits & 0x007FFFFF) | 0x3F800000, jnp.float32)
    nl_m = _D0 + m * (_D1 + m * (_D2 + m * (_D3 + m * _D4)))
    nl = e * (-_LN2) + nl_m
    return jnp.where(q > 0.0, nl, 100.0)


@functools.partial(
    pl.kernel,
    mesh=_mesh,
    out_type=jax.ShapeDtypeStruct((_NW, 4, _L), jnp.float32),
    scratch_types=[
        pltpu.VMEM((2, _CHUNK_R, _C), jnp.float32),
        pltpu.VMEM((2, _CHUNK_R, _C), jnp.float32),
        pltpu.VMEM((2, _CHUNK_R, _C), jnp.float32),
        pltpu.VMEM((4, _L), jnp.float32),
        pltpu.SemaphoreType.DMA,
        pltpu.SemaphoreType.DMA,
    ],
    compiler_params=pltpu.CompilerParams(use_tc_tiling_on_sc=True),
)
def _pass1(pred_h, gt_h, mask_h, out_h, pb, gb, mb, acc, s0, s1):
    wid = lax.axis_index("c") * _NS + lax.axis_index("s")
    base = wid * _ROWS_W
    sems = (s0, s1)

    def start(c):
        b = c % 2
        row = base + c * _CHUNK_R
        return [
            pltpu.async_copy(h.at[pl.ds(row, _CHUNK_R), :], buf.at[b], sems[b])
            for h, buf in ((pred_h, pb), (gt_h, gb), (mask_h, mb))
        ]

    zeros = jnp.zeros((_L,), jnp.float32)
    carry = (zeros, zeros, zeros, zeros)
    pend = start(0)
    for c in range(_NCHUNKS):
        nxt = start(c + 1) if c + 1 < _NCHUNKS else None
        for h in pend:
            h.wait()
        b = c % 2

        def row_body(r, cr0, b=b):
            def slice_body(i, cr, b=b, r=r):
                ps, ts, pc, mc = cr
                for u in range(_UNROLL):
                    sl = pl.ds(i * (_L * _UNROLL) + u * _L, _L)
                    p = pb[b, r, sl]
                    g = gb[b, r, sl]
                    m = mb[b, r, sl]
                    q = jnp.where(g > 0.5, p, 1.0 - p)
                    l = _neg_log_clipped(q)
                    gm = g * m
                    ps = ps + l * gm
                    ts = ts + l * m
                    pc = pc + gm
                    mc = mc + m
                return (ps, ts, pc, mc)

            return lax.fori_loop(0, _C // (_L * _UNROLL), slice_body, cr0)

        carry = lax.fori_loop(0, _CHUNK_R, row_body, carry)
        pend = nxt
    ps, ts, pc, mc = carry
    acc[0, :] = ps
    acc[1, :] = ts
    acc[2, :] = pc
    acc[3, :] = mc
    pltpu.sync_copy(acc, out_h.at[wid])


def _fold8(x):
    acc = x[0:8]
    for i in range(1, _TC_BLOCK_R // 8):
        acc = acc + x[i * 8:(i + 1) * 8]
    return acc


def _tc_body(p_ref, g_ref, m_ref, o_ref):
    p = p_ref[...]
    g = g_ref[...]
    m = m_ref[...]
    q = jnp.where(g > 0.5, p, 1.0 - p)
    bits = lax.bitcast_convert_type(q, jnp.int32)
    e = (bits >> 23).astype(jnp.float32)
    mant = lax.bitcast_convert_type((bits & 0x007FFFFF) | 0x3F800000, jnp.float32)
    nl_m = _D0 + mant * (_D1 + mant * (_D2 + mant * (_D3 + mant * _D4)))
    l = jnp.where(q > 0.0, e * (-_LN2) + nl_m, 100.0)
    gm = g * m
    w8 = jnp.full((8, _TC_BLOCK_R), 0.125, jnp.float32)
    o_ref[...] = jnp.concatenate(
        [w8 @ (l * gm), w8 @ (l * m), w8 @ gm, w8 @ m], axis=0
    )


def _thr_body(p_ref, g_ref, m_ref, t_ref, o_ref):
    t = t_ref[0, 0]
    p = p_ref[...]
    g = g_ref[...]
    m = m_ref[...]
    q = jnp.where(g > 0.5, p, 1.0 - p)
    bits = lax.bitcast_convert_type(q, jnp.int32)
    e = (bits >> 23).astype(jnp.float32)
    mant = lax.bitcast_convert_type((bits & 0x007FFFFF) | 0x3F800000, jnp.float32)
    nl_m = _D0 + mant * (_D1 + mant * (_D2 + mant * (_D3 + mant * _D4)))
    l = jnp.where(q > 0.0, e * (-_LN2) + nl_m, 100.0)
    v = l * ((1.0 - g) * m)
    above = (v > t).astype(jnp.float32)
    o_ref[...] = jnp.concatenate([_fold8(above), _fold8(v * above)], axis=0)


_thr_grid = _R // _TC_BLOCK_R
_thr_pass = pl.pallas_call(
    _thr_body,
    grid=(_thr_grid,),
    in_specs=[pl.BlockSpec((_TC_BLOCK_R, _C), lambda i: (i, 0))] * 3
    + [pl.BlockSpec(memory_space=pltpu.SMEM)],
    out_specs=pl.BlockSpec((16, _C), lambda i: (i, 0)),
    out_shape=jax.ShapeDtypeStruct((_thr_grid * 16, _C), jnp.float32),
)


def _count_sum_above(pf, gf, mf, thresh):
    parts = _thr_pass(pf, gf, mf, thresh.reshape(1, 1))
    r = parts.reshape(_thr_grid, 2, 8, _C).sum(axis=(0, 2, 3))
    return r[0], r[1]


_tc_grid = _T_ROWS // _TC_BLOCK_R
_tc_off = _S_ROWS // _TC_BLOCK_R
_tc_pass = pl.pallas_call(
    _tc_body,
    grid=(_tc_grid,),
    in_specs=[pl.BlockSpec((_TC_BLOCK_R, _C), lambda i: (i + _tc_off, 0))] * 3,
    out_specs=pl.BlockSpec((32, _C), lambda i: (i, 0)),
    out_shape=jax.ShapeDtypeStruct((_tc_grid * 32, _C), jnp.float32),
)


def kernel(pred, gt, mask):
    pf = pred.reshape(_R, _C)
    gf = gt.reshape(_R, _C)
    mf = mask.reshape(_R, _C)
    parts = _pass1(pf, gf, mf)
    tc_parts = _tc_pass(pf, gf, mf)
    t = parts.sum(axis=(0, 2)) + tc_parts.reshape(_tc_grid, 4, 8, _C).sum(
        axis=(0, 2, 3)
    )
    pos_loss, tot_loss, no_pos, mask_cnt = t[0], t[1], t[2], t[3]
    neg_loss = tot_loss - pos_loss
    neg_cnt = mask_cnt - no_pos
    no_neg = jnp.minimum(no_pos * 3.0, neg_cnt)

    def _topk_all(_):
        return neg_loss

    def _topk_search(_):
        k = no_neg

        def body(st):
            lo, hi = st
            mid = (lo + hi) // 2
            c, _ = _count_sum_above(pf, gf, mf, lax.bitcast_convert_type(mid, jnp.float32))
            ok = c <= k
            return (jnp.where(ok, lo, mid + 1), jnp.where(ok, mid, hi))

        lo, _ = lax.while_loop(
            lambda st: st[0] < st[1],
            body,
            (jnp.int32(0), jnp.int32(0x42C80000)),
        )
        tstar = lax.bitcast_convert_type(lo, jnp.float32)
        c, s = _count_sum_above(pf, gf, mf, tstar)
        return s + (k - c) * tstar

    topk = lax.cond(no_neg >= neg_cnt, _topk_all, _topk_search, None)
    return (pos_loss + topk) / (no_neg + no_pos + 1e-6)
